# baseline (speedup 1.0000x reference)
import functools

import jax
import jax.numpy as jnp
import numpy as np
from jax.experimental import pallas as pl

_MASK_RATIO = 0.6
_BLOCK_RATIO = 0.02
_ADJUST_RATIO = 0.1

_consts_cache = {}


def _get_consts(B, P):
    if (B, P) in _consts_cache:
        return _consts_cache[(B, P)]
    block_size = int(_BLOCK_RATIO * P)
    block_fraction = (_MASK_RATIO - _ADJUST_RATIO) / block_size
    num_centers = round(P * block_fraction)
    with jax.ensure_compile_time_eval():
        k1, k2 = jax.random.split(jax.random.key(42))
        ru1 = np.asarray(jax.random.uniform(k1, (B, P), dtype=jnp.float32))
        ru2 = np.asarray(jax.random.uniform(k2, (B, P), dtype=jnp.float32))
    ci = np.argsort(ru1, axis=-1, kind="stable")[:, :num_centers].astype(np.int32)
    perm = np.argsort(ru2, axis=-1, kind="stable")
    rk = np.empty((B, P), np.int32)
    rk[np.arange(B)[:, None], perm] = np.arange(P, dtype=np.int32)[None, :]
    _consts_cache[(B, P)] = (ci, rk)
    return ci, rk


def _body(pts_ref, sel_ref, rk_ref, out_ref, *, NC, K, NM, P):
    p = pts_ref[0]
    sel = sel_ref[0]
    rk = rk_ref[0]

    px, py, pz = p[0:1, :], p[1:2, :], p[2:3, :]
    s2 = px * px + py * py + pz * pz
    sx, sy, sz = sel[:, 0:1], sel[:, 1:2], sel[:, 2:3]
    s1 = sx * sx + sy * sy + sz * sz
    bf = lambda v: v.astype(jnp.bfloat16).astype(jnp.float32)
    dot = bf(sx) * bf(px) + bf(sy) * bf(py) + bf(sz) * bf(pz)
    d = (s1 + s2) - 2.0 * dot
    d = jnp.maximum(d, 0.0)
    keys = jax.lax.bitcast_convert_type(d, jnp.int32)

    kf = jnp.float32(K)

    def count_le(mid):
        return jnp.sum(jnp.where(keys <= mid, 1.0, 0.0), axis=1, keepdims=True)

    def t_step(mid, lo, clo, hi, chi):
        cnt = count_le(mid)
        ge = cnt >= kf
        lo = jnp.where(ge, lo, mid + 1)
        clo = jnp.where(ge, clo, cnt)
        hi = jnp.where(ge, mid, hi)
        chi = jnp.where(ge, cnt, chi)
        return lo, clo, hi, chi

    def t_interp(lo, clo, hi, chi):
        vlo = jax.lax.bitcast_convert_type(lo, jnp.float32)
        vhi = jax.lax.bitcast_convert_type(hi, jnp.float32)
        frac = (kf - clo) / jnp.maximum(chi - clo, 1.0)
        vmid = vlo + frac * (vhi - vlo)
        mid = jax.lax.bitcast_convert_type(vmid, jnp.int32)
        mid = jnp.clip(mid, lo, jnp.maximum(hi - 1, lo))
        return t_step(mid, lo, clo, hi, chi)

    def t_body(carry):
        i, lo, clo, hi, chi = carry
        lo, clo, hi, chi = t_interp(lo, clo, hi, chi)
        lo, clo, hi, chi = t_interp(lo, clo, hi, chi)
        lo, clo, hi, chi = t_interp(lo, clo, hi, chi)
        mid = lo + ((hi - lo) >> 1)
        lo, clo, hi, chi = t_step(mid, lo, clo, hi, chi)
        return i + 1, lo, clo, hi, chi

    def t_cond(carry):
        i, lo, _, hi, _ = carry
        return (i < 32) & jnp.any(lo < hi)

    lo0 = jnp.zeros((NC, 1), jnp.int32)
    clo0 = jnp.zeros((NC, 1), jnp.float32)
    hi0 = jnp.max(keys, axis=1, keepdims=True)
    chi0 = jnp.full((NC, 1), jnp.float32(P))
    _, t, n_less, _, n_le = jax.lax.while_loop(
        t_cond, t_body, (jnp.int32(0), lo0, clo0, hi0, chi0))

    extra = kf - n_less
    n_eq = n_le - n_less
    jidx = jax.lax.broadcasted_iota(jnp.int32, (NC, P), 1)

    def cov_fast():
        return jnp.any(keys <= t, axis=0, keepdims=True).astype(jnp.int32)

    def cov_tie():
        eq = keys == t

        def jbody(_, lh):
            lo, hi = lh
            mid = lo + ((hi - lo) >> 1)
            cnt = jnp.sum(jnp.where(eq & (jidx <= mid), 1.0, 0.0),
                          axis=1, keepdims=True)
            ge = cnt >= extra
            return jnp.where(ge, lo, mid + 1), jnp.where(ge, mid, hi)

        jlo0 = jnp.zeros((NC, 1), jnp.int32)
        jhi0 = jnp.full((NC, 1), jnp.int32(P - 1))
        jthr, _ = jax.lax.fori_loop(0, 13, jbody, (jlo0, jhi0))
        covered = (keys < t) | (eq & (jidx <= jthr))
        return jnp.any(covered, axis=0, keepdims=True).astype(jnp.int32)

    tie_any = jnp.any(n_eq > extra)
    cov = jax.lax.cond(tie_any, cov_tie, cov_fast) != 0

    T = jnp.sum(jnp.where(cov, 1.0, 0.0), axis=1, keepdims=True)
    A = jnp.float32(NM) - T

    uncov = ~cov

    def f_count(mid):
        return jnp.sum(jnp.where(uncov & (rk <= mid), 1.0, 0.0),
                       axis=1, keepdims=True)

    def f_step(mid, lo, clo, hi, chi):
        cnt = f_count(mid)
        ge = cnt >= A
        lo = jnp.where(ge, lo, mid + 1)
        clo = jnp.where(ge, clo, cnt)
        hi = jnp.where(ge, mid, hi)
        chi = jnp.where(ge, cnt, chi)
        return lo, clo, hi, chi

    def f_body(carry):
        i, lo, clo, hi, chi = carry
        span = (hi - lo + 1).astype(jnp.float32)
        est = (A - clo) * span / jnp.maximum(chi - clo, 1.0)
        mid = lo - 1 + est.astype(jnp.int32)
        mid = jnp.clip(mid, lo, jnp.maximum(hi - 1, lo))
        lo, clo, hi, chi = f_step(mid, lo, clo, hi, chi)
        mid = lo + ((hi - lo) >> 1)
        lo, clo, hi, chi = f_step(mid, lo, clo, hi, chi)
        return i + 1, lo, clo, hi, chi

    def f_cond(carry):
        i, lo, _, hi, _ = carry
        return (i < 16) & jnp.any(lo < hi)

    flo0 = jnp.zeros((1, 1), jnp.int32)
    fclo0 = jnp.zeros((1, 1), jnp.float32)
    fhi0 = jnp.full((1, 1), jnp.int32(P - 1))
    fchi0 = jnp.float32(P) - T
    _, m, _, _, _ = jax.lax.while_loop(
        f_cond, f_body, (jnp.int32(0), flo0, fclo0, fhi0, fchi0))

    out_ref[0] = (cov | (uncov & (rk <= m))).astype(jnp.int32)


def kernel(centers):
    B, P, _ = centers.shape
    num_masks = round(_MASK_RATIO * P)
    block_size = int(_BLOCK_RATIO * P)
    num_centers = round(P * (_MASK_RATIO - _ADJUST_RATIO) / block_size)
    ci, rk = _get_consts(B, P)

    sel = jnp.take_along_axis(centers, jnp.asarray(ci)[..., None], axis=1)
    nc_pad = -(-num_centers // 8) * 8
    if nc_pad != num_centers:
        sel = jnp.concatenate(
            [sel, jnp.broadcast_to(sel[:, :1, :],
                                   (B, nc_pad - num_centers, 3))], axis=1)
    pts = centers.transpose(0, 2, 1)
    rk3 = jnp.asarray(rk).reshape(B, 1, P)

    body = functools.partial(_body, NC=nc_pad, K=block_size,
                             NM=num_masks, P=P)
    out = pl.pallas_call(
        body,
        grid=(B,),
        in_specs=[
            pl.BlockSpec((1, 3, P), lambda b: (b, 0, 0)),
            pl.BlockSpec((1, nc_pad, 3), lambda b: (b, 0, 0)),
            pl.BlockSpec((1, 1, P), lambda b: (b, 0, 0)),
        ],
        out_specs=pl.BlockSpec((1, 1, P), lambda b: (b, 0, 0)),
        out_shape=jax.ShapeDtypeStruct((B, 1, P), jnp.int32),
    )(pts, sel, rk3)
    return out.reshape(B, P).astype(bool)


# speedup vs baseline: 1.5180x; 1.5180x over previous
import functools

import jax
import jax.numpy as jnp
import numpy as np
from jax.experimental import pallas as pl

_MASK_RATIO = 0.6
_BLOCK_RATIO = 0.02
_ADJUST_RATIO = 0.1

_consts_cache = {}


def _get_consts(B, P):
    if (B, P) in _consts_cache:
        return _consts_cache[(B, P)]
    block_size = int(_BLOCK_RATIO * P)
    block_fraction = (_MASK_RATIO - _ADJUST_RATIO) / block_size
    num_centers = round(P * block_fraction)
    with jax.ensure_compile_time_eval():
        k1, k2 = jax.random.split(jax.random.key(42))
        ru1 = np.asarray(jax.random.uniform(k1, (B, P), dtype=jnp.float32))
        ru2 = np.asarray(jax.random.uniform(k2, (B, P), dtype=jnp.float32))
    ci = np.argsort(ru1, axis=-1, kind="stable")[:, :num_centers].astype(np.int32)
    perm = np.argsort(ru2, axis=-1, kind="stable")
    rk = np.empty((B, P), np.int32)
    rk[np.arange(B)[:, None], perm] = np.arange(P, dtype=np.int32)[None, :]
    _consts_cache[(B, P)] = (ci, rk)
    return ci, rk


def _body(pts_ref, sel_ref, rk_ref, out_ref, *, NC, K, NM, P):
    p = pts_ref[0]
    sel = sel_ref[0]
    rk = rk_ref[0]

    px, py, pz = p[0:1, :], p[1:2, :], p[2:3, :]
    s2 = px * px + py * py + pz * pz
    sx, sy, sz = sel[:, 0:1], sel[:, 1:2], sel[:, 2:3]
    s1 = sx * sx + sy * sy + sz * sz
    bf = lambda v: v.astype(jnp.bfloat16).astype(jnp.float32)
    dot = bf(sx) * bf(px) + bf(sy) * bf(py) + bf(sz) * bf(pz)
    d = (s1 + s2) - 2.0 * dot
    d = jnp.maximum(d, 0.0)
    keys = jax.lax.bitcast_convert_type(d, jnp.int32)

    ki = jnp.int32(K)

    def count_le(mid):
        return -jnp.sum((keys - mid - 1) >> 31, axis=1, keepdims=True)

    def t_body(carry):
        i, lo, clo, hi, chi = carry
        mid = lo + ((hi - lo) >> 1)
        cnt = count_le(mid)
        ge = cnt >= ki
        lo = jnp.where(ge, lo, mid + 1)
        clo = jnp.where(ge, clo, cnt)
        hi = jnp.where(ge, mid, hi)
        chi = jnp.where(ge, cnt, chi)
        return i + 1, lo, clo, hi, chi

    def t_cond(carry):
        i, lo, _, hi, _ = carry
        return (i < 32) & jnp.any(lo < hi)

    lo0 = jnp.min(keys, axis=1, keepdims=True)
    clo0 = jnp.zeros((NC, 1), jnp.int32)
    hi0 = jnp.max(keys, axis=1, keepdims=True)
    chi0 = jnp.full((NC, 1), jnp.int32(P))
    _, t, n_less, _, n_le = jax.lax.while_loop(
        t_cond, t_body, (jnp.int32(0), lo0, clo0, hi0, chi0))

    extra = ki - n_less
    n_eq = n_le - n_less
    jidx = jax.lax.broadcasted_iota(jnp.int32, (NC, P), 1)

    def cov_fast():
        return jnp.any(keys <= t, axis=0, keepdims=True).astype(jnp.int32)

    def cov_tie():
        eq = keys == t

        def jbody(_, lh):
            lo, hi = lh
            mid = lo + ((hi - lo) >> 1)
            cnt = jnp.sum(jnp.where(eq & (jidx <= mid), 1, 0),
                          axis=1, keepdims=True)
            ge = cnt >= extra
            return jnp.where(ge, lo, mid + 1), jnp.where(ge, mid, hi)

        jlo0 = jnp.zeros((NC, 1), jnp.int32)
        jhi0 = jnp.full((NC, 1), jnp.int32(P - 1))
        jthr, _ = jax.lax.fori_loop(0, 13, jbody, (jlo0, jhi0))
        covered = (keys < t) | (eq & (jidx <= jthr))
        return jnp.any(covered, axis=0, keepdims=True).astype(jnp.int32)

    tie_any = jnp.any(n_eq > extra)
    cov = jax.lax.cond(tie_any, cov_tie, cov_fast) != 0

    T = jnp.sum(jnp.where(cov, 1, 0), axis=1, keepdims=True)
    A = jnp.int32(NM) - T

    uncov = ~cov
    um = jnp.where(uncov, -1, 0)

    def f_count(mid):
        return jnp.sum(um & ((rk - mid - 1) >> 31), axis=1, keepdims=True) * -1

    def f_step(mid, lo, clo, hi, chi):
        cnt = f_count(mid)
        ge = cnt >= A
        lo = jnp.where(ge, lo, mid + 1)
        clo = jnp.where(ge, clo, cnt)
        hi = jnp.where(ge, mid, hi)
        chi = jnp.where(ge, cnt, chi)
        return lo, clo, hi, chi

    def f_body(carry):
        i, lo, clo, hi, chi = carry
        span = (hi - lo + 1).astype(jnp.float32)
        est = (A - clo).astype(jnp.float32) * span \
            / jnp.maximum(chi - clo, 1).astype(jnp.float32)
        mid = lo - 1 + est.astype(jnp.int32)
        mid = jnp.clip(mid, lo, jnp.maximum(hi - 1, lo))
        lo, clo, hi, chi = f_step(mid, lo, clo, hi, chi)
        mid = lo + ((hi - lo) >> 1)
        lo, clo, hi, chi = f_step(mid, lo, clo, hi, chi)
        return i + 1, lo, clo, hi, chi

    def f_cond(carry):
        i, lo, _, hi, _ = carry
        return (i < 16) & jnp.any(lo < hi)

    flo0 = jnp.zeros((1, 1), jnp.int32)
    fclo0 = jnp.zeros((1, 1), jnp.int32)
    fhi0 = jnp.full((1, 1), jnp.int32(P - 1))
    fchi0 = jnp.int32(P) - T
    _, m, _, _, _ = jax.lax.while_loop(
        f_cond, f_body, (jnp.int32(0), flo0, fclo0, fhi0, fchi0))

    out_ref[0] = (cov | (uncov & (rk <= m))).astype(jnp.int32)


def kernel(centers):
    B, P, _ = centers.shape
    num_masks = round(_MASK_RATIO * P)
    block_size = int(_BLOCK_RATIO * P)
    num_centers = round(P * (_MASK_RATIO - _ADJUST_RATIO) / block_size)
    ci, rk = _get_consts(B, P)

    sel = jnp.take_along_axis(centers, jnp.asarray(ci)[..., None], axis=1)
    nc_pad = -(-num_centers // 8) * 8
    if nc_pad != num_centers:
        sel = jnp.concatenate(
            [sel, jnp.broadcast_to(sel[:, :1, :],
                                   (B, nc_pad - num_centers, 3))], axis=1)
    pts = centers.transpose(0, 2, 1)
    rk3 = jnp.asarray(rk).reshape(B, 1, P)

    body = functools.partial(_body, NC=nc_pad, K=block_size,
                             NM=num_masks, P=P)
    out = pl.pallas_call(
        body,
        grid=(B,),
        in_specs=[
            pl.BlockSpec((1, 3, P), lambda b: (b, 0, 0)),
            pl.BlockSpec((1, nc_pad, 3), lambda b: (b, 0, 0)),
            pl.BlockSpec((1, 1, P), lambda b: (b, 0, 0)),
        ],
        out_specs=pl.BlockSpec((1, 1, P), lambda b: (b, 0, 0)),
        out_shape=jax.ShapeDtypeStruct((B, 1, P), jnp.int32),
    )(pts, sel, rk3)
    return out.reshape(B, P).astype(bool)


# speedup vs baseline: 1.7500x; 1.1529x over previous
import functools

import jax
import jax.numpy as jnp
import numpy as np
from jax.experimental import pallas as pl

_MASK_RATIO = 0.6
_BLOCK_RATIO = 0.02
_ADJUST_RATIO = 0.1

_consts_cache = {}


def _get_consts(B, P):
    if (B, P) in _consts_cache:
        return _consts_cache[(B, P)]
    block_size = int(_BLOCK_RATIO * P)
    block_fraction = (_MASK_RATIO - _ADJUST_RATIO) / block_size
    num_centers = round(P * block_fraction)
    with jax.ensure_compile_time_eval():
        k1, k2 = jax.random.split(jax.random.key(42))
        ru1 = np.asarray(jax.random.uniform(k1, (B, P), dtype=jnp.float32))
        ru2 = np.asarray(jax.random.uniform(k2, (B, P), dtype=jnp.float32))
    ci = np.argsort(ru1, axis=-1, kind="stable")[:, :num_centers].astype(np.int32)
    perm = np.argsort(ru2, axis=-1, kind="stable")
    rk = np.empty((B, P), np.int32)
    rk[np.arange(B)[:, None], perm] = np.arange(P, dtype=np.int32)[None, :]
    _consts_cache[(B, P)] = (ci, rk)
    return ci, rk


def _body(pts_ref, sel_ref, rk_ref, out_ref, *, NC, K, NM, P):
    p = pts_ref[0]
    sel = sel_ref[0]
    rk = rk_ref[0]

    px, py, pz = p[0:1, :], p[1:2, :], p[2:3, :]
    s2 = px * px + py * py + pz * pz
    sx, sy, sz = sel[:, 0:1], sel[:, 1:2], sel[:, 2:3]
    s1 = sx * sx + sy * sy + sz * sz
    bf = lambda v: v.astype(jnp.bfloat16).astype(jnp.float32)
    dot = bf(sx) * bf(px) + bf(sy) * bf(py) + bf(sz) * bf(pz)
    d = (s1 + s2) - 2.0 * dot
    d = jnp.maximum(d, 0.0)
    keys = jax.lax.bitcast_convert_type(d, jnp.int32)

    ki = jnp.int32(K)

    def count_le(mid):
        return -jnp.sum((keys - mid - 1) >> 31, axis=1, keepdims=True)

    def t_body(_, carry):
        lo, clo, hi, chi = carry
        mid = lo + ((hi - lo) >> 1)
        cnt = count_le(mid)
        ge = cnt >= ki
        lo = jnp.where(ge, lo, mid + 1)
        clo = jnp.where(ge, clo, cnt)
        hi = jnp.where(ge, mid, hi)
        chi = jnp.where(ge, cnt, chi)
        return lo, clo, hi, chi

    lo0 = jnp.zeros((NC, 1), jnp.int32)
    clo0 = jnp.zeros((NC, 1), jnp.int32)
    hi0 = jnp.full((NC, 1), jnp.int32(0x7F7FFFFF))
    chi0 = jnp.full((NC, 1), jnp.int32(P))
    t, n_less, _, n_le = jax.lax.fori_loop(
        0, 31, t_body, (lo0, clo0, hi0, chi0))

    extra = ki - n_less
    n_eq = n_le - n_less
    jidx = jax.lax.broadcasted_iota(jnp.int32, (NC, P), 1)

    def cov_fast():
        return jnp.any(keys <= t, axis=0, keepdims=True).astype(jnp.int32)

    def cov_tie():
        eq = keys == t

        def jbody(_, lh):
            lo, hi = lh
            mid = lo + ((hi - lo) >> 1)
            cnt = jnp.sum(jnp.where(eq & (jidx <= mid), 1, 0),
                          axis=1, keepdims=True)
            ge = cnt >= extra
            return jnp.where(ge, lo, mid + 1), jnp.where(ge, mid, hi)

        jlo0 = jnp.zeros((NC, 1), jnp.int32)
        jhi0 = jnp.full((NC, 1), jnp.int32(P - 1))
        jthr, _ = jax.lax.fori_loop(0, 13, jbody, (jlo0, jhi0))
        covered = (keys < t) | (eq & (jidx <= jthr))
        return jnp.any(covered, axis=0, keepdims=True).astype(jnp.int32)

    tie_any = jnp.any(n_eq > extra)
    cov = jax.lax.cond(tie_any, cov_tie, cov_fast) != 0

    T = jnp.sum(jnp.where(cov, 1, 0), axis=1, keepdims=True)
    A = jnp.int32(NM) - T

    uncov = ~cov
    um = jnp.where(uncov, -1, 0)

    def f_count(mid):
        return jnp.sum(um & ((rk - mid - 1) >> 31), axis=1, keepdims=True) * -1

    def f_step(mid, lo, clo, hi, chi):
        cnt = f_count(mid)
        ge = cnt >= A
        lo = jnp.where(ge, lo, mid + 1)
        clo = jnp.where(ge, clo, cnt)
        hi = jnp.where(ge, mid, hi)
        chi = jnp.where(ge, cnt, chi)
        return lo, clo, hi, chi

    def f_body(carry):
        i, lo, clo, hi, chi = carry
        span = (hi - lo + 1).astype(jnp.float32)
        est = (A - clo).astype(jnp.float32) * span \
            / jnp.maximum(chi - clo, 1).astype(jnp.float32)
        mid = lo - 1 + est.astype(jnp.int32)
        mid = jnp.clip(mid, lo, jnp.maximum(hi - 1, lo))
        lo, clo, hi, chi = f_step(mid, lo, clo, hi, chi)
        mid = lo + ((hi - lo) >> 1)
        lo, clo, hi, chi = f_step(mid, lo, clo, hi, chi)
        return i + 1, lo, clo, hi, chi

    def f_cond(carry):
        i, lo, _, hi, _ = carry
        return (i < 16) & jnp.any(lo < hi)

    flo0 = jnp.zeros((1, 1), jnp.int32)
    fclo0 = jnp.zeros((1, 1), jnp.int32)
    fhi0 = jnp.full((1, 1), jnp.int32(P - 1))
    fchi0 = jnp.int32(P) - T
    _, m, _, _, _ = jax.lax.while_loop(
        f_cond, f_body, (jnp.int32(0), flo0, fclo0, fhi0, fchi0))

    out_ref[0] = (cov | (uncov & (rk <= m))).astype(jnp.int32)


def kernel(centers):
    B, P, _ = centers.shape
    num_masks = round(_MASK_RATIO * P)
    block_size = int(_BLOCK_RATIO * P)
    num_centers = round(P * (_MASK_RATIO - _ADJUST_RATIO) / block_size)
    ci, rk = _get_consts(B, P)

    sel = jnp.take_along_axis(centers, jnp.asarray(ci)[..., None], axis=1)
    nc_pad = -(-num_centers // 8) * 8
    if nc_pad != num_centers:
        sel = jnp.concatenate(
            [sel, jnp.broadcast_to(sel[:, :1, :],
                                   (B, nc_pad - num_centers, 3))], axis=1)
    pts = centers.transpose(0, 2, 1)
    rk3 = jnp.asarray(rk).reshape(B, 1, P)

    body = functools.partial(_body, NC=nc_pad, K=block_size,
                             NM=num_masks, P=P)
    out = pl.pallas_call(
        body,
        grid=(B,),
        in_specs=[
            pl.BlockSpec((1, 3, P), lambda b: (b, 0, 0)),
            pl.BlockSpec((1, nc_pad, 3), lambda b: (b, 0, 0)),
            pl.BlockSpec((1, 1, P), lambda b: (b, 0, 0)),
        ],
        out_specs=pl.BlockSpec((1, 1, P), lambda b: (b, 0, 0)),
        out_shape=jax.ShapeDtypeStruct((B, 1, P), jnp.int32),
    )(pts, sel, rk3)
    return out.reshape(B, P).astype(bool)
